# baseline (speedup 1.0000x reference)
import functools

import jax
import jax.numpy as jnp
from jax import lax
from jax.experimental import pallas as pl
from jax.experimental.pallas import tpu as pltpu
from jax.experimental.pallas import tpu_sc as plsc

_N_SEG = 10000
_N = 320000
_D = 128

_NC = 2
_NS = 16
_NW = _NC * _NS
_ROWS_PER_TILE = _N // _NW
_CHUNK = 128
_NFULL = _ROWS_PER_TILE // _CHUNK
_TAIL = _ROWS_PER_TILE - _NFULL * _CHUNK
_OUT_ROWS_PER_TILE = 624
_OUT_REMAINDER = _N_SEG - _NS * _OUT_ROWS_PER_TILE


_mesh = plsc.VectorSubcoreMesh(core_axis_name="c", subcore_axis_name="s")


@functools.partial(
    pl.kernel,
    mesh=_mesh,
    out_type=jax.ShapeDtypeStruct((_NC, _N_SEG, _D), jnp.float32),
    scratch_types=[
        pltpu.VMEM((_CHUNK,), jnp.int32),
        pltpu.VMEM((_CHUNK,), jnp.int32),
        pltpu.VMEM((_CHUNK,), jnp.int32),
        pltpu.VMEM((_TAIL,), jnp.int32),
        pltpu.VMEM((_CHUNK, _D), jnp.float32),
        pltpu.VMEM((_CHUNK, _D), jnp.float32),
        pltpu.VMEM((_CHUNK, _D), jnp.float32),
        pltpu.SemaphoreType.DMA,
        pltpu.SemaphoreType.DMA,
        pltpu.SemaphoreType.DMA,
        pltpu.SemaphoreType.DMA,
        pltpu.SemaphoreType.DMA,
        pltpu.SemaphoreType.DMA,
        pltpu.VMEM_SHARED((_N_SEG, _D), jnp.float32),
    ],
)
def _seg_partials(v_hbm, batch_hbm, part_hbm,
                  idx0, idx1, idx2, idx_t,
                  rows0, rows1, rows2,
                  in0, in1, in2, sc0, sc1, sc2, acc_sh):
    c = lax.axis_index("c")
    s = lax.axis_index("s")
    wid = s * _NC + c
    idx = (idx0, idx1, idx2)
    rows = (rows0, rows1, rows2)
    sem_in = (in0, in1, in2)
    sem_sc = (sc0, sc1, sc2)

    base0 = wid * _ROWS_PER_TILE

    def _load(i, b):
        base = pl.multiple_of(base0 + i * _CHUNK, 8)
        pltpu.async_copy(batch_hbm.at[pl.ds(base, _CHUNK)], idx[b], sem_in[b])
        pltpu.async_copy(v_hbm.at[pl.ds(base, _CHUNK)], rows[b], sem_in[b])

    def _wait_load(i, b):
        base = pl.multiple_of(base0 + i * _CHUNK, 8)
        pltpu.make_async_copy(
            batch_hbm.at[pl.ds(base, _CHUNK)], idx[b], sem_in[b]).wait()
        pltpu.make_async_copy(
            v_hbm.at[pl.ds(base, _CHUNK)], rows[b], sem_in[b]).wait()

    def _wait_scatter(b):
        pltpu.make_async_copy(
            rows[b], acc_sh.at[idx[b]], sem_sc[b]).wait()

    _load(0, 0)
    _load(1, 1)

    def _zfill(i, carry):
        for j in range(_D // 16):
            rows2[i, pl.ds(j * 16, 16)] = jnp.zeros((16,), jnp.float32)
        return carry

    lax.fori_loop(0, _CHUNK, _zfill, 0)
    zbase = pl.multiple_of(s * _OUT_ROWS_PER_TILE, 8)
    pltpu.sync_copy(rows2, acc_sh.at[pl.ds(zbase, _CHUNK)])
    pltpu.sync_copy(rows2, acc_sh.at[pl.ds(zbase + _CHUNK, _CHUNK)])
    pltpu.sync_copy(rows2, acc_sh.at[pl.ds(zbase + 2 * _CHUNK, _CHUNK)])
    pltpu.sync_copy(rows2, acc_sh.at[pl.ds(zbase + 3 * _CHUNK, _CHUNK)])
    pltpu.sync_copy(
        rows2.at[pl.ds(0, _OUT_ROWS_PER_TILE - 4 * _CHUNK)],
        acc_sh.at[pl.ds(zbase + 4 * _CHUNK, _OUT_ROWS_PER_TILE - 4 * _CHUNK)],
    )

    @pl.when(s == _NS - 1)
    def _zero_tail():
        pltpu.sync_copy(
            rows2.at[pl.ds(0, _OUT_REMAINDER)],
            acc_sh.at[pl.ds(_NS * _OUT_ROWS_PER_TILE, _OUT_REMAINDER)],
        )

    plsc.subcore_barrier()

    def _step(i, b):
        _wait_load(i, b)
        pltpu.async_copy(rows[b], acc_sh.at[idx[b]], sem_sc[b], add=True)

        @pl.when(i >= 1)
        def _retire():
            _wait_scatter((b + 2) % 3)

        @pl.when(i + 2 < _NFULL)
        def _prefetch():
            _load(i + 2, (b + 2) % 3)

    def _outer(i3, carry):
        for b in range(3):
            _step(i3 * 3 + b, b)
        return carry

    lax.fori_loop(0, _NFULL // 3, _outer, 0)
    _wait_scatter(2)

    tbase = pl.multiple_of(base0 + _NFULL * _CHUNK, 8)
    pltpu.sync_copy(batch_hbm.at[pl.ds(tbase, _TAIL)], idx_t)
    pltpu.sync_copy(v_hbm.at[pl.ds(tbase, _TAIL)], rows0.at[pl.ds(0, _TAIL)])
    pltpu.sync_copy(rows0.at[pl.ds(0, _TAIL)], acc_sh.at[idx_t], add=True)

    plsc.subcore_barrier()

    obase = pl.multiple_of(s * _OUT_ROWS_PER_TILE, 8)
    pltpu.sync_copy(
        acc_sh.at[pl.ds(obase, _OUT_ROWS_PER_TILE)],
        part_hbm.at[c, pl.ds(obase, _OUT_ROWS_PER_TILE)],
    )

    @pl.when(s == _NS - 1)
    def _write_tail():
        pltpu.sync_copy(
            acc_sh.at[pl.ds(_NS * _OUT_ROWS_PER_TILE, _OUT_REMAINDER)],
            part_hbm.at[c, pl.ds(_NS * _OUT_ROWS_PER_TILE, _OUT_REMAINDER)],
        )


_BLK = 1000


def _combine_body(u_ref, p0_ref, p1_ref, o_ref):
    o_ref[...] = u_ref[...] + p0_ref[...] + p1_ref[...]


def kernel(u, v, batch):
    part = _seg_partials(v, batch)
    out = pl.pallas_call(
        _combine_body,
        grid=(_N_SEG // _BLK,),
        in_specs=[
            pl.BlockSpec((_BLK, _D), lambda i: (i, 0)),
            pl.BlockSpec((_BLK, _D), lambda i: (i, 0)),
            pl.BlockSpec((_BLK, _D), lambda i: (i, 0)),
        ],
        out_specs=pl.BlockSpec((_BLK, _D), lambda i: (i, 0)),
        out_shape=jax.ShapeDtypeStruct((_N_SEG, _D), jnp.float32),
    )(u, part[0], part[1])
    return out


# speedup vs baseline: 1.0294x; 1.0294x over previous
import functools

import jax
import jax.numpy as jnp
from jax import lax
from jax.experimental import pallas as pl
from jax.experimental.pallas import tpu as pltpu
from jax.experimental.pallas import tpu_sc as plsc

_N_SEG = 10000
_N = 320000
_D = 128

_NC = 2
_NS = 16
_NW = _NC * _NS
_ROWS_PER_TILE = _N // _NW
_CHUNK = 128
_NFULL = _ROWS_PER_TILE // _CHUNK
_TAIL = _ROWS_PER_TILE - _NFULL * _CHUNK
_OUT_ROWS_PER_TILE = 624
_OUT_REMAINDER = _N_SEG - _NS * _OUT_ROWS_PER_TILE


_mesh = plsc.VectorSubcoreMesh(core_axis_name="c", subcore_axis_name="s")


@functools.partial(
    pl.kernel,
    mesh=_mesh,
    out_type=jax.ShapeDtypeStruct((_NC, _N_SEG, _D), jnp.float32),
    scratch_types=[
        pltpu.VMEM((_CHUNK,), jnp.int32),
        pltpu.VMEM((_CHUNK,), jnp.int32),
        pltpu.VMEM((_CHUNK,), jnp.int32),
        pltpu.VMEM((_TAIL,), jnp.int32),
        pltpu.VMEM((_CHUNK, _D), jnp.float32),
        pltpu.VMEM((_CHUNK, _D), jnp.float32),
        pltpu.VMEM((_CHUNK, _D), jnp.float32),
        pltpu.SemaphoreType.DMA,
        pltpu.SemaphoreType.DMA,
        pltpu.SemaphoreType.DMA,
        pltpu.SemaphoreType.DMA,
        pltpu.SemaphoreType.DMA,
        pltpu.SemaphoreType.DMA,
        pltpu.VMEM_SHARED((_N_SEG, _D), jnp.float32),
    ],
)
def _seg_partials(v_hbm, batch_hbm, part_hbm,
                  idx0, idx1, idx2, idx_t,
                  rows0, rows1, rows2,
                  in0, in1, in2, sc0, sc1, sc2, acc_sh):
    c = lax.axis_index("c")
    s = lax.axis_index("s")
    wid = s * _NC + c
    idx = (idx0, idx1, idx2)
    rows = (rows0, rows1, rows2)
    sem_in = (in0, in1, in2)
    sem_sc = (sc0, sc1, sc2)

    base0 = wid * _ROWS_PER_TILE

    def _load(i, b):
        base = pl.multiple_of(base0 + i * _CHUNK, 8)
        pltpu.async_copy(batch_hbm.at[pl.ds(base, _CHUNK)], idx[b], sem_in[b])
        pltpu.async_copy(v_hbm.at[pl.ds(base, _CHUNK)], rows[b], sem_in[b])

    def _wait_load(i, b):
        base = pl.multiple_of(base0 + i * _CHUNK, 8)
        pltpu.make_async_copy(
            batch_hbm.at[pl.ds(base, _CHUNK)], idx[b], sem_in[b]).wait()
        pltpu.make_async_copy(
            v_hbm.at[pl.ds(base, _CHUNK)], rows[b], sem_in[b]).wait()

    def _wait_scatter(b):
        pltpu.make_async_copy(
            rows[b], acc_sh.at[idx[b]], sem_sc[b]).wait()

    _load(0, 0)
    _load(1, 1)

    def _zfill(i, carry):
        for j in range(_D // 16):
            rows2[i, pl.ds(j * 16, 16)] = jnp.zeros((16,), jnp.float32)
        return carry

    lax.fori_loop(0, _CHUNK, _zfill, 0)
    zbase = pl.multiple_of(s * _OUT_ROWS_PER_TILE, 8)
    pltpu.sync_copy(rows2, acc_sh.at[pl.ds(zbase, _CHUNK)])
    pltpu.sync_copy(rows2, acc_sh.at[pl.ds(zbase + _CHUNK, _CHUNK)])
    pltpu.sync_copy(rows2, acc_sh.at[pl.ds(zbase + 2 * _CHUNK, _CHUNK)])
    pltpu.sync_copy(rows2, acc_sh.at[pl.ds(zbase + 3 * _CHUNK, _CHUNK)])
    pltpu.sync_copy(
        rows2.at[pl.ds(0, _OUT_ROWS_PER_TILE - 4 * _CHUNK)],
        acc_sh.at[pl.ds(zbase + 4 * _CHUNK, _OUT_ROWS_PER_TILE - 4 * _CHUNK)],
    )

    @pl.when(s == _NS - 1)
    def _zero_tail():
        pltpu.sync_copy(
            rows2.at[pl.ds(0, _OUT_REMAINDER)],
            acc_sh.at[pl.ds(_NS * _OUT_ROWS_PER_TILE, _OUT_REMAINDER)],
        )

    plsc.subcore_barrier()

    def _step(i, b):
        _wait_load(i, b)
        pltpu.async_copy(rows[b], acc_sh.at[idx[b]], sem_sc[b],
                         add=True).wait()

        @pl.when(i + 2 < _NFULL)
        def _prefetch():
            _load(i + 2, b)

    def _outer(i2, carry):
        for b in range(2):
            _step(i2 * 2 + b, b)
        return carry

    lax.fori_loop(0, _NFULL // 2, _outer, 0)

    tbase = pl.multiple_of(base0 + _NFULL * _CHUNK, 8)
    pltpu.sync_copy(batch_hbm.at[pl.ds(tbase, _TAIL)], idx_t)
    pltpu.sync_copy(v_hbm.at[pl.ds(tbase, _TAIL)], rows0.at[pl.ds(0, _TAIL)])
    pltpu.sync_copy(rows0.at[pl.ds(0, _TAIL)], acc_sh.at[idx_t], add=True)

    plsc.subcore_barrier()

    obase = pl.multiple_of(s * _OUT_ROWS_PER_TILE, 8)
    pltpu.sync_copy(
        acc_sh.at[pl.ds(obase, _OUT_ROWS_PER_TILE)],
        part_hbm.at[c, pl.ds(obase, _OUT_ROWS_PER_TILE)],
    )

    @pl.when(s == _NS - 1)
    def _write_tail():
        pltpu.sync_copy(
            acc_sh.at[pl.ds(_NS * _OUT_ROWS_PER_TILE, _OUT_REMAINDER)],
            part_hbm.at[c, pl.ds(_NS * _OUT_ROWS_PER_TILE, _OUT_REMAINDER)],
        )


_BLK = 1000


def _combine_body(u_ref, p0_ref, p1_ref, o_ref):
    o_ref[...] = u_ref[...] + p0_ref[...] + p1_ref[...]


def kernel(u, v, batch):
    part = _seg_partials(v, batch)
    out = pl.pallas_call(
        _combine_body,
        grid=(_N_SEG // _BLK,),
        in_specs=[
            pl.BlockSpec((_BLK, _D), lambda i: (i, 0)),
            pl.BlockSpec((_BLK, _D), lambda i: (i, 0)),
            pl.BlockSpec((_BLK, _D), lambda i: (i, 0)),
        ],
        out_specs=pl.BlockSpec((_BLK, _D), lambda i: (i, 0)),
        out_shape=jax.ShapeDtypeStruct((_N_SEG, _D), jnp.float32),
    )(u, part[0], part[1])
    return out


# speedup vs baseline: 1.0667x; 1.0362x over previous
import functools

import jax
import jax.numpy as jnp
from jax import lax
from jax.experimental import pallas as pl
from jax.experimental.pallas import tpu as pltpu
from jax.experimental.pallas import tpu_sc as plsc

_N_SEG = 10000
_N = 320000
_D = 128

_NC = 2
_NS = 16
_NW = _NC * _NS
_ROWS_PER_TILE = _N // _NW
_CHUNK = 128
_NFULL = _ROWS_PER_TILE // _CHUNK
_TAIL = _ROWS_PER_TILE - _NFULL * _CHUNK
_OUT_ROWS_PER_TILE = 624
_OUT_REMAINDER = _N_SEG - _NS * _OUT_ROWS_PER_TILE


_mesh = plsc.VectorSubcoreMesh(core_axis_name="c", subcore_axis_name="s")


@functools.partial(
    pl.kernel,
    mesh=_mesh,
    out_type=jax.ShapeDtypeStruct((_NC, _N_SEG, _D), jnp.float32),
    scratch_types=[
        pltpu.VMEM((_CHUNK,), jnp.int32),
        pltpu.VMEM((_CHUNK,), jnp.int32),
        pltpu.VMEM((_CHUNK,), jnp.int32),
        pltpu.VMEM((_TAIL,), jnp.int32),
        pltpu.VMEM((_CHUNK, _D), jnp.float32),
        pltpu.VMEM((_CHUNK, _D), jnp.float32),
        pltpu.VMEM((_CHUNK, _D), jnp.float32),
        pltpu.SemaphoreType.DMA,
        pltpu.SemaphoreType.DMA,
        pltpu.SemaphoreType.DMA,
        pltpu.SemaphoreType.DMA,
        pltpu.SemaphoreType.DMA,
        pltpu.SemaphoreType.DMA,
        pltpu.VMEM_SHARED((_N_SEG, _D), jnp.float32),
    ],
)
def _seg_partials(v_hbm, batch_hbm, u_hbm, part_hbm,
                  idx0, idx1, idx2, idx_t,
                  rows0, rows1, rows2,
                  in0, in1, in2, sc0, sc1, sc2, acc_sh):
    c = lax.axis_index("c")
    s = lax.axis_index("s")
    wid = s * _NC + c
    idx = (idx0, idx1, idx2)
    rows = (rows0, rows1, rows2)
    sem_in = (in0, in1, in2)
    sem_sc = (sc0, sc1, sc2)

    base0 = wid * _ROWS_PER_TILE

    def _load(i, b):
        base = pl.multiple_of(base0 + i * _CHUNK, 8)
        pltpu.async_copy(batch_hbm.at[pl.ds(base, _CHUNK)], idx[b], sem_in[b])
        pltpu.async_copy(v_hbm.at[pl.ds(base, _CHUNK)], rows[b], sem_in[b])

    def _wait_load(i, b):
        base = pl.multiple_of(base0 + i * _CHUNK, 8)
        pltpu.make_async_copy(
            batch_hbm.at[pl.ds(base, _CHUNK)], idx[b], sem_in[b]).wait()
        pltpu.make_async_copy(
            v_hbm.at[pl.ds(base, _CHUNK)], rows[b], sem_in[b]).wait()

    def _wait_scatter(b):
        pltpu.make_async_copy(
            rows[b], acc_sh.at[idx[b]], sem_sc[b]).wait()

    _load(0, 0)
    _load(1, 1)

    zbase = pl.multiple_of(s * _OUT_ROWS_PER_TILE, 8)

    @pl.when(c == 0)
    def _seed_u():
        pltpu.sync_copy(u_hbm.at[pl.ds(zbase, _OUT_ROWS_PER_TILE)],
                        acc_sh.at[pl.ds(zbase, _OUT_ROWS_PER_TILE)])

        @pl.when(s == _NS - 1)
        def _seed_u_tail():
            pltpu.sync_copy(
                u_hbm.at[pl.ds(_NS * _OUT_ROWS_PER_TILE, _OUT_REMAINDER)],
                acc_sh.at[pl.ds(_NS * _OUT_ROWS_PER_TILE, _OUT_REMAINDER)],
            )

    @pl.when(c == 1)
    def _seed_zero():
        def _zfill(i, carry):
            for j in range(_D // 16):
                rows2[i, pl.ds(j * 16, 16)] = jnp.zeros((16,), jnp.float32)
            return carry

        lax.fori_loop(0, _CHUNK, _zfill, 0)
        pltpu.sync_copy(rows2, acc_sh.at[pl.ds(zbase, _CHUNK)])
        pltpu.sync_copy(rows2, acc_sh.at[pl.ds(zbase + _CHUNK, _CHUNK)])
        pltpu.sync_copy(rows2, acc_sh.at[pl.ds(zbase + 2 * _CHUNK, _CHUNK)])
        pltpu.sync_copy(rows2, acc_sh.at[pl.ds(zbase + 3 * _CHUNK, _CHUNK)])
        pltpu.sync_copy(
            rows2.at[pl.ds(0, _OUT_ROWS_PER_TILE - 4 * _CHUNK)],
            acc_sh.at[pl.ds(zbase + 4 * _CHUNK,
                            _OUT_ROWS_PER_TILE - 4 * _CHUNK)],
        )

        @pl.when(s == _NS - 1)
        def _zero_tail():
            pltpu.sync_copy(
                rows2.at[pl.ds(0, _OUT_REMAINDER)],
                acc_sh.at[pl.ds(_NS * _OUT_ROWS_PER_TILE, _OUT_REMAINDER)],
            )

    plsc.subcore_barrier()

    def _step(i, b):
        _wait_load(i, b)
        pltpu.async_copy(rows[b], acc_sh.at[idx[b]], sem_sc[b],
                         add=True).wait()

        @pl.when(i + 2 < _NFULL)
        def _prefetch():
            _load(i + 2, b)

    def _outer(i2, carry):
        for b in range(2):
            _step(i2 * 2 + b, b)
        return carry

    lax.fori_loop(0, _NFULL // 2, _outer, 0)

    tbase = pl.multiple_of(base0 + _NFULL * _CHUNK, 8)
    pltpu.sync_copy(batch_hbm.at[pl.ds(tbase, _TAIL)], idx_t)
    pltpu.sync_copy(v_hbm.at[pl.ds(tbase, _TAIL)], rows0.at[pl.ds(0, _TAIL)])
    pltpu.sync_copy(rows0.at[pl.ds(0, _TAIL)], acc_sh.at[idx_t], add=True)

    plsc.subcore_barrier()

    obase = pl.multiple_of(s * _OUT_ROWS_PER_TILE, 8)
    pltpu.sync_copy(
        acc_sh.at[pl.ds(obase, _OUT_ROWS_PER_TILE)],
        part_hbm.at[c, pl.ds(obase, _OUT_ROWS_PER_TILE)],
    )

    @pl.when(s == _NS - 1)
    def _write_tail():
        pltpu.sync_copy(
            acc_sh.at[pl.ds(_NS * _OUT_ROWS_PER_TILE, _OUT_REMAINDER)],
            part_hbm.at[c, pl.ds(_NS * _OUT_ROWS_PER_TILE, _OUT_REMAINDER)],
        )


_BLK = 1000


def _combine_body(p_ref, o_ref):
    o_ref[...] = p_ref[0] + p_ref[1]


def kernel(u, v, batch):
    part = _seg_partials(v, batch, u)
    out = pl.pallas_call(
        _combine_body,
        grid=(_N_SEG // _BLK,),
        in_specs=[
            pl.BlockSpec((2, _BLK, _D), lambda i: (0, i, 0)),
        ],
        out_specs=pl.BlockSpec((_BLK, _D), lambda i: (i, 0)),
        out_shape=jax.ShapeDtypeStruct((_N_SEG, _D), jnp.float32),
    )(part)
    return out


# speedup vs baseline: 1.0816x; 1.0140x over previous
import functools

import jax
import jax.numpy as jnp
from jax import lax
from jax.experimental import pallas as pl
from jax.experimental.pallas import tpu as pltpu
from jax.experimental.pallas import tpu_sc as plsc

_N_SEG = 10000
_N = 320000
_D = 128

_NC = 2
_NS = 16
_NW = _NC * _NS
_ROWS_PER_TILE = _N // _NW
_CHUNK = 128
_NFULL = _ROWS_PER_TILE // _CHUNK
_TAIL = _ROWS_PER_TILE - _NFULL * _CHUNK
_OUT_ROWS_PER_TILE = 624
_OUT_REMAINDER = _N_SEG - _NS * _OUT_ROWS_PER_TILE


_mesh = plsc.VectorSubcoreMesh(core_axis_name="c", subcore_axis_name="s")


@functools.partial(
    pl.kernel,
    mesh=_mesh,
    out_type=jax.ShapeDtypeStruct((_NC, _N_SEG, _D), jnp.float32),
    scratch_types=[
        pltpu.VMEM((_CHUNK,), jnp.int32),
        pltpu.VMEM((_CHUNK,), jnp.int32),
        pltpu.VMEM((_CHUNK,), jnp.int32),
        pltpu.VMEM((_TAIL,), jnp.int32),
        pltpu.VMEM((_CHUNK, _D), jnp.float32),
        pltpu.VMEM((_CHUNK, _D), jnp.float32),
        pltpu.VMEM((_CHUNK, _D), jnp.float32),
        pltpu.SemaphoreType.DMA,
        pltpu.SemaphoreType.DMA,
        pltpu.SemaphoreType.DMA,
        pltpu.SemaphoreType.DMA,
        pltpu.SemaphoreType.DMA,
        pltpu.SemaphoreType.DMA,
        pltpu.VMEM_SHARED((_N_SEG, _D), jnp.float32),
    ],
)
def _seg_partials(v_hbm, batch_hbm, u_hbm, part_hbm,
                  idx0, idx1, idx2, idx_t,
                  rows0, rows1, rows2,
                  in0, in1, in2, sc0, sc1, sc2, acc_sh):
    c = lax.axis_index("c")
    s = lax.axis_index("s")
    wid = s * _NC + c
    idx = (idx0, idx1, idx2)
    rows = (rows0, rows1, rows2)
    sem_in = (in0, in1, in2)
    sem_sc = (sc0, sc1, sc2)

    base0 = wid * _ROWS_PER_TILE

    def _load(i, b):
        base = pl.multiple_of(base0 + i * _CHUNK, 8)
        pltpu.async_copy(batch_hbm.at[pl.ds(base, _CHUNK)], idx[b], sem_in[b])
        pltpu.async_copy(v_hbm.at[pl.ds(base, _CHUNK)], rows[b], sem_in[b])

    def _wait_load(i, b):
        base = pl.multiple_of(base0 + i * _CHUNK, 8)
        pltpu.make_async_copy(
            batch_hbm.at[pl.ds(base, _CHUNK)], idx[b], sem_in[b]).wait()
        pltpu.make_async_copy(
            v_hbm.at[pl.ds(base, _CHUNK)], rows[b], sem_in[b]).wait()

    def _wait_scatter(b):
        pltpu.make_async_copy(
            rows[b], acc_sh.at[idx[b]], sem_sc[b]).wait()

    _load(0, 0)
    _load(1, 1)

    zbase = pl.multiple_of(s * _OUT_ROWS_PER_TILE, 8)

    @pl.when(c == 0)
    def _seed_u():
        pltpu.sync_copy(u_hbm.at[pl.ds(zbase, _OUT_ROWS_PER_TILE)],
                        acc_sh.at[pl.ds(zbase, _OUT_ROWS_PER_TILE)])

        @pl.when(s == _NS - 1)
        def _seed_u_tail():
            pltpu.sync_copy(
                u_hbm.at[pl.ds(_NS * _OUT_ROWS_PER_TILE, _OUT_REMAINDER)],
                acc_sh.at[pl.ds(_NS * _OUT_ROWS_PER_TILE, _OUT_REMAINDER)],
            )

    @pl.when(c == 1)
    def _seed_zero():
        def _zfill(i, carry):
            for j in range(_D // 16):
                rows2[i, pl.ds(j * 16, 16)] = jnp.zeros((16,), jnp.float32)
            return carry

        lax.fori_loop(0, _CHUNK, _zfill, 0)
        pltpu.sync_copy(rows2, acc_sh.at[pl.ds(zbase, _CHUNK)])
        pltpu.sync_copy(rows2, acc_sh.at[pl.ds(zbase + _CHUNK, _CHUNK)])
        pltpu.sync_copy(rows2, acc_sh.at[pl.ds(zbase + 2 * _CHUNK, _CHUNK)])
        pltpu.sync_copy(rows2, acc_sh.at[pl.ds(zbase + 3 * _CHUNK, _CHUNK)])
        pltpu.sync_copy(
            rows2.at[pl.ds(0, _OUT_ROWS_PER_TILE - 4 * _CHUNK)],
            acc_sh.at[pl.ds(zbase + 4 * _CHUNK,
                            _OUT_ROWS_PER_TILE - 4 * _CHUNK)],
        )

        @pl.when(s == _NS - 1)
        def _zero_tail():
            pltpu.sync_copy(
                rows2.at[pl.ds(0, _OUT_REMAINDER)],
                acc_sh.at[pl.ds(_NS * _OUT_ROWS_PER_TILE, _OUT_REMAINDER)],
            )

    plsc.subcore_barrier()

    def _step(i, b):
        _wait_load(i, b)
        pltpu.async_copy(rows[b], acc_sh.at[idx[b]], sem_sc[b],
                         add=True).wait()

        @pl.when(i + 2 < _NFULL)
        def _prefetch():
            _load(i + 2, b)

    def _outer(i2, carry):
        for b in range(2):
            _step(i2 * 2 + b, b)
        return carry

    lax.fori_loop(0, _NFULL // 2, _outer, 0)

    tbase = pl.multiple_of(base0 + _NFULL * _CHUNK, 8)
    pltpu.sync_copy(batch_hbm.at[pl.ds(tbase, _TAIL)], idx_t)
    pltpu.sync_copy(v_hbm.at[pl.ds(tbase, _TAIL)], rows0.at[pl.ds(0, _TAIL)])
    pltpu.sync_copy(rows0.at[pl.ds(0, _TAIL)], acc_sh.at[idx_t], add=True)

    plsc.subcore_barrier()

    obase = pl.multiple_of(s * _OUT_ROWS_PER_TILE, 8)
    pltpu.sync_copy(
        acc_sh.at[pl.ds(obase, _OUT_ROWS_PER_TILE)],
        part_hbm.at[c, pl.ds(obase, _OUT_ROWS_PER_TILE)],
    )

    @pl.when(s == _NS - 1)
    def _write_tail():
        pltpu.sync_copy(
            acc_sh.at[pl.ds(_NS * _OUT_ROWS_PER_TILE, _OUT_REMAINDER)],
            part_hbm.at[c, pl.ds(_NS * _OUT_ROWS_PER_TILE, _OUT_REMAINDER)],
        )


_BLK = 2000


def _combine_body(p_ref, o_ref):
    o_ref[...] = p_ref[0] + p_ref[1]


def kernel(u, v, batch):
    part = _seg_partials(v, batch, u)
    out = pl.pallas_call(
        _combine_body,
        grid=(_N_SEG // _BLK,),
        in_specs=[
            pl.BlockSpec((2, _BLK, _D), lambda i: (0, i, 0)),
        ],
        out_specs=pl.BlockSpec((_BLK, _D), lambda i: (i, 0)),
        out_shape=jax.ShapeDtypeStruct((_N_SEG, _D), jnp.float32),
    )(part)
    return out


# speedup vs baseline: 1.0959x; 1.0132x over previous
import functools

import jax
import jax.numpy as jnp
from jax import lax
from jax.experimental import pallas as pl
from jax.experimental.pallas import tpu as pltpu
from jax.experimental.pallas import tpu_sc as plsc

_N_SEG = 10000
_N = 320000
_D = 128

_NC = 2
_NS = 16
_NW = _NC * _NS
_ROWS_PER_TILE = _N // _NW
_CHUNK = 128
_NFULL = _ROWS_PER_TILE // _CHUNK
_TAIL = _ROWS_PER_TILE - _NFULL * _CHUNK
_OUT_ROWS_PER_TILE = 624
_OUT_REMAINDER = _N_SEG - _NS * _OUT_ROWS_PER_TILE


_mesh = plsc.VectorSubcoreMesh(core_axis_name="c", subcore_axis_name="s")


@functools.partial(
    pl.kernel,
    mesh=_mesh,
    out_type=jax.ShapeDtypeStruct((_NC, _N_SEG, _D), jnp.float32),
    scratch_types=[
        pltpu.VMEM((_CHUNK,), jnp.int32),
        pltpu.VMEM((_CHUNK,), jnp.int32),
        pltpu.VMEM((_CHUNK,), jnp.int32),
        pltpu.VMEM((_TAIL,), jnp.int32),
        pltpu.VMEM((_CHUNK, _D), jnp.float32),
        pltpu.VMEM((_CHUNK, _D), jnp.float32),
        pltpu.VMEM((_CHUNK, _D), jnp.float32),
        pltpu.SemaphoreType.DMA,
        pltpu.SemaphoreType.DMA,
        pltpu.SemaphoreType.DMA,
        pltpu.SemaphoreType.DMA,
        pltpu.SemaphoreType.DMA,
        pltpu.SemaphoreType.DMA,
        pltpu.VMEM_SHARED((_N_SEG, _D), jnp.float32),
    ],
)
def _seg_partials(v_hbm, batch_hbm, u_hbm, part_hbm,
                  idx0, idx1, idx2, idx_t,
                  rows0, rows1, rows2,
                  in0, in1, in2, sc0, sc1, sc2, acc_sh):
    c = lax.axis_index("c")
    s = lax.axis_index("s")
    wid = s * _NC + c
    idx = (idx0, idx1, idx2)
    rows = (rows0, rows1, rows2)
    sem_in = (in0, in1, in2)
    sem_sc = (sc0, sc1, sc2)

    base0 = wid * _ROWS_PER_TILE

    def _load(i, b):
        base = pl.multiple_of(base0 + i * _CHUNK, 8)
        pltpu.async_copy(batch_hbm.at[pl.ds(base, _CHUNK)], idx[b], sem_in[b])
        pltpu.async_copy(v_hbm.at[pl.ds(base, _CHUNK)], rows[b], sem_in[b])

    def _wait_load(i, b):
        base = pl.multiple_of(base0 + i * _CHUNK, 8)
        pltpu.make_async_copy(
            batch_hbm.at[pl.ds(base, _CHUNK)], idx[b], sem_in[b]).wait()
        pltpu.make_async_copy(
            v_hbm.at[pl.ds(base, _CHUNK)], rows[b], sem_in[b]).wait()

    def _wait_scatter(b):
        pltpu.make_async_copy(
            rows[b], acc_sh.at[idx[b]], sem_sc[b]).wait()

    _load(0, 0)
    _load(1, 1)

    zbase = pl.multiple_of(s * _OUT_ROWS_PER_TILE, 8)

    @pl.when(c == 0)
    def _seed_u():
        pltpu.sync_copy(u_hbm.at[pl.ds(zbase, _OUT_ROWS_PER_TILE)],
                        acc_sh.at[pl.ds(zbase, _OUT_ROWS_PER_TILE)])

        @pl.when(s == _NS - 1)
        def _seed_u_tail():
            pltpu.sync_copy(
                u_hbm.at[pl.ds(_NS * _OUT_ROWS_PER_TILE, _OUT_REMAINDER)],
                acc_sh.at[pl.ds(_NS * _OUT_ROWS_PER_TILE, _OUT_REMAINDER)],
            )

    @pl.when(c == 1)
    def _seed_zero():
        def _zfill(i, carry):
            for j in range(_D // 16):
                rows2[i, pl.ds(j * 16, 16)] = jnp.zeros((16,), jnp.float32)
            return carry

        lax.fori_loop(0, _CHUNK, _zfill, 0)
        pltpu.sync_copy(rows2, acc_sh.at[pl.ds(zbase, _CHUNK)])
        pltpu.sync_copy(rows2, acc_sh.at[pl.ds(zbase + _CHUNK, _CHUNK)])
        pltpu.sync_copy(rows2, acc_sh.at[pl.ds(zbase + 2 * _CHUNK, _CHUNK)])
        pltpu.sync_copy(rows2, acc_sh.at[pl.ds(zbase + 3 * _CHUNK, _CHUNK)])
        pltpu.sync_copy(
            rows2.at[pl.ds(0, _OUT_ROWS_PER_TILE - 4 * _CHUNK)],
            acc_sh.at[pl.ds(zbase + 4 * _CHUNK,
                            _OUT_ROWS_PER_TILE - 4 * _CHUNK)],
        )

        @pl.when(s == _NS - 1)
        def _zero_tail():
            pltpu.sync_copy(
                rows2.at[pl.ds(0, _OUT_REMAINDER)],
                acc_sh.at[pl.ds(_NS * _OUT_ROWS_PER_TILE, _OUT_REMAINDER)],
            )

    plsc.subcore_barrier()

    def _step(i, b):
        _wait_load(i, b)
        pltpu.async_copy(rows[b], acc_sh.at[idx[b]], sem_sc[b],
                         add=True).wait()

        @pl.when(i + 2 < _NFULL)
        def _prefetch():
            _load(i + 2, b)

    def _outer(i2, carry):
        for b in range(2):
            _step(i2 * 2 + b, b)
        return carry

    lax.fori_loop(0, _NFULL // 2, _outer, 0)

    tbase = pl.multiple_of(base0 + _NFULL * _CHUNK, 8)
    pltpu.sync_copy(batch_hbm.at[pl.ds(tbase, _TAIL)], idx_t)
    pltpu.sync_copy(v_hbm.at[pl.ds(tbase, _TAIL)], rows0.at[pl.ds(0, _TAIL)])
    pltpu.sync_copy(rows0.at[pl.ds(0, _TAIL)], acc_sh.at[idx_t], add=True)

    plsc.subcore_barrier()

    obase = pl.multiple_of(s * _OUT_ROWS_PER_TILE, 8)
    pltpu.sync_copy(
        acc_sh.at[pl.ds(obase, _OUT_ROWS_PER_TILE)],
        part_hbm.at[c, pl.ds(obase, _OUT_ROWS_PER_TILE)],
    )

    @pl.when(s == _NS - 1)
    def _write_tail():
        pltpu.sync_copy(
            acc_sh.at[pl.ds(_NS * _OUT_ROWS_PER_TILE, _OUT_REMAINDER)],
            part_hbm.at[c, pl.ds(_NS * _OUT_ROWS_PER_TILE, _OUT_REMAINDER)],
        )


_BLK = 5000


def _combine_body(p_ref, o_ref):
    o_ref[...] = p_ref[0] + p_ref[1]


def kernel(u, v, batch):
    part = _seg_partials(v, batch, u)
    out = pl.pallas_call(
        _combine_body,
        grid=(_N_SEG // _BLK,),
        in_specs=[
            pl.BlockSpec((2, _BLK, _D), lambda i: (0, i, 0)),
        ],
        out_specs=pl.BlockSpec((_BLK, _D), lambda i: (i, 0)),
        out_shape=jax.ShapeDtypeStruct((_N_SEG, _D), jnp.float32),
    )(part)
    return out
